# baseline (speedup 1.0000x reference)
import functools

import jax
import jax.numpy as jnp
from jax.experimental import pallas as pl
from jax.experimental.pallas import tpu as pltpu

_B = 16384
_F = 13
_EPS = 1e-5

_NT = (((1,), (1,)), ((), ()))


def _fused_kernel(
    x_ref, wemb_ref, bemb_ref, w1_ref, b1_ref, g1_ref, be1_ref,
    w2_ref, b2_ref, g2_ref, be2_ref, w3_ref, b3_ref,
    ww_ref, bw_ref, wc_ref, bc_ref,
    out_ref,
):
    xT = jnp.transpose(x_ref[...])

    embT = jnp.dot(wemb_ref[...], xT) + bemb_ref[...]
    dots = jnp.sum(embT * embT[0:1, :], axis=1, keepdims=True)
    nrm2 = jnp.sum(embT * embT, axis=1, keepdims=True)
    nrm = jnp.sqrt(nrm2)
    q = nrm[0:1, 0:1]
    dist = 1.0 - dots / (nrm * q + 1e-12)
    drow = jnp.transpose(dist)
    dself = jnp.broadcast_to(dist, (_F, _F))
    dk = jnp.broadcast_to(drow, (_F, _F))
    kk = jax.lax.broadcasted_iota(jnp.int32, (_F, _F), 1)
    jj = jax.lax.broadcasted_iota(jnp.int32, (_F, _F), 0)
    before = (dk < dself) | ((dk == dself) & (kk < jj))
    rank = jnp.sum(before.astype(jnp.float32), axis=1, keepdims=True)
    j0 = jax.lax.broadcasted_iota(jnp.int32, (_F, 1), 0)
    wide_m = ((rank >= float(_F - 6)) | (j0 == 0)).astype(jnp.float32)
    deep_m = 1.0 - wide_m
    lt = (kk < jj).astype(jnp.float32)
    pos_w = jnp.dot(lt, wide_m)
    pos_d = jnp.dot(lt, deep_m)
    cols7 = jax.lax.broadcasted_iota(jnp.int32, (_F, 7), 1).astype(
        jnp.float32)
    cols6 = jax.lax.broadcasted_iota(jnp.int32, (_F, 6), 1).astype(
        jnp.float32)
    s_wide = jnp.where(pos_w == cols7, wide_m, 0.0)
    s_deep = jnp.where(pos_d == cols6, deep_m, 0.0)
    w1dT = jax.lax.dot_general(w1_ref[...], s_deep, _NT)
    wwfT = jax.lax.dot_general(ww_ref[...], s_wide, _NT)

    h1T = jnp.dot(w1dT, xT) + b1_ref[...]
    m1 = jnp.sum(h1T, axis=1, keepdims=True) * (1.0 / _B)
    v1 = jnp.sum(h1T * h1T, axis=1, keepdims=True) * (1.0 / _B) - m1 * m1
    sc1 = g1_ref[...] * jax.lax.rsqrt(v1 + _EPS)
    sh1 = be1_ref[...] - m1 * sc1
    a1T = jnp.maximum(h1T * sc1 + sh1, 0.0)

    h2T = jnp.dot(w2_ref[...], a1T) + b2_ref[...]
    m2 = jnp.sum(h2T, axis=1, keepdims=True) * (1.0 / _B)
    v2 = jnp.sum(h2T * h2T, axis=1, keepdims=True) * (1.0 / _B) - m2 * m2
    sc2 = g2_ref[...] * jax.lax.rsqrt(v2 + _EPS)
    sh2 = be2_ref[...] - m2 * sc2
    a2T = jnp.maximum(h2T * sc2 + sh2, 0.0)

    dnnT = jnp.dot(w3_ref[...], a2T) + b3_ref[...]

    wideT = jnp.dot(wwfT, xT) + bw_ref[...]
    gluT = dnnT * jax.nn.sigmoid(wideT)
    logitT = jnp.dot(wc_ref[...], gluT) + bc_ref[0, 0]
    out_ref[...] = jax.nn.sigmoid(logitT)


@functools.partial(jax.jit)
def kernel(x, W_emb, b_emb, W1, b1, g1, be1, W2, b2, g2, be2, W3, b3,
           Ww, bw, Wc, bc):
    f32 = jnp.float32
    col = lambda v: v.reshape(-1, 1).astype(f32)
    args = (
        x.astype(f32),
        W_emb.astype(f32), col(b_emb),
        W1.astype(f32), col(b1), col(g1), col(be1),
        W2.astype(f32), col(b2), col(g2), col(be2),
        W3.astype(f32), col(b3),
        Ww.astype(f32), col(bw),
        Wc.astype(f32), bc.reshape(1, 1).astype(f32),
    )
    in_specs = [
        pl.BlockSpec((_B, _F), lambda: (0, 0)),
        pl.BlockSpec((_F, _F), lambda: (0, 0)),
        pl.BlockSpec((_F, 1), lambda: (0, 0)),
        pl.BlockSpec((64, 6), lambda: (0, 0)),
        pl.BlockSpec((64, 1), lambda: (0, 0)),
        pl.BlockSpec((64, 1), lambda: (0, 0)),
        pl.BlockSpec((64, 1), lambda: (0, 0)),
        pl.BlockSpec((32, 64), lambda: (0, 0)),
        pl.BlockSpec((32, 1), lambda: (0, 0)),
        pl.BlockSpec((32, 1), lambda: (0, 0)),
        pl.BlockSpec((32, 1), lambda: (0, 0)),
        pl.BlockSpec((64, 32), lambda: (0, 0)),
        pl.BlockSpec((64, 1), lambda: (0, 0)),
        pl.BlockSpec((64, 7), lambda: (0, 0)),
        pl.BlockSpec((64, 1), lambda: (0, 0)),
        pl.BlockSpec((1, 64), lambda: (0, 0)),
        pl.BlockSpec(memory_space=pltpu.SMEM),
    ]
    return pl.pallas_call(
        _fused_kernel,
        in_specs=in_specs,
        out_specs=pl.BlockSpec((1, _B), lambda: (0, 0)),
        out_shape=jax.ShapeDtypeStruct((1, _B), f32),
    )(*args).reshape(_B, 1)


# speedup vs baseline: 1.1613x; 1.1613x over previous
import functools

import jax
import jax.numpy as jnp
from jax.experimental import pallas as pl
from jax.experimental.pallas import tpu as pltpu

_B = 16384
_F = 13
_EPS = 1e-5

_NT = (((1,), (1,)), ((), ()))

_R_WEMB = 0
_R_W1 = 16
_R_W2 = 80
_R_W3 = 112
_R_WW = 176
_R_WC = 240
_R_BEMB = 248
_R_B1 = 264
_R_G1 = 328
_R_BE1 = 392
_R_B2 = 456
_R_G2 = 488
_R_BE2 = 520
_R_B3 = 552
_R_BW = 616
_R_BC = 680
_ROWS = 688


def _fused_kernel(x_ref, p_ref, out_ref):
    xT = jnp.transpose(x_ref[...])

    embT = (jnp.dot(p_ref[_R_WEMB:_R_WEMB + _F, 0:_F], xT)
            + p_ref[_R_BEMB:_R_BEMB + _F, 0:1])
    dots = jnp.sum(embT * embT[0:1, :], axis=1, keepdims=True)
    nrm2 = jnp.sum(embT * embT, axis=1, keepdims=True)
    nrm = jnp.sqrt(nrm2)
    q = nrm[0:1, 0:1]
    dist = 1.0 - dots / (nrm * q + 1e-12)
    drow = jnp.transpose(dist)
    dself = jnp.broadcast_to(dist, (_F, _F))
    dk = jnp.broadcast_to(drow, (_F, _F))
    kk = jax.lax.broadcasted_iota(jnp.int32, (_F, _F), 1)
    jj = jax.lax.broadcasted_iota(jnp.int32, (_F, _F), 0)
    before = (dk < dself) | ((dk == dself) & (kk < jj))
    rank = jnp.sum(before.astype(jnp.float32), axis=1, keepdims=True)
    j0 = jax.lax.broadcasted_iota(jnp.int32, (_F, 1), 0)
    wide_m = ((rank >= float(_F - 6)) | (j0 == 0)).astype(jnp.float32)
    deep_m = 1.0 - wide_m
    lt = (kk < jj).astype(jnp.float32)
    pos_w = jnp.dot(lt, wide_m)
    pos_d = jnp.dot(lt, deep_m)
    cols7 = jax.lax.broadcasted_iota(jnp.int32, (_F, 7), 1).astype(
        jnp.float32)
    cols6 = jax.lax.broadcasted_iota(jnp.int32, (_F, 6), 1).astype(
        jnp.float32)
    s_wide = jnp.where(pos_w == cols7, wide_m, 0.0)
    s_deep = jnp.where(pos_d == cols6, deep_m, 0.0)
    w1dT = jax.lax.dot_general(
        p_ref[_R_W1:_R_W1 + 64, 0:6], s_deep, _NT)
    wwfT = jax.lax.dot_general(
        p_ref[_R_WW:_R_WW + 64, 0:7], s_wide, _NT)

    h1T = jnp.dot(w1dT, xT) + p_ref[_R_B1:_R_B1 + 64, 0:1]
    m1 = jnp.sum(h1T, axis=1, keepdims=True) * (1.0 / _B)
    v1 = jnp.sum(h1T * h1T, axis=1, keepdims=True) * (1.0 / _B) - m1 * m1
    sc1 = p_ref[_R_G1:_R_G1 + 64, 0:1] * jax.lax.rsqrt(v1 + _EPS)
    sh1 = p_ref[_R_BE1:_R_BE1 + 64, 0:1] - m1 * sc1
    a1T = jnp.maximum(h1T * sc1 + sh1, 0.0)

    h2T = (jnp.dot(p_ref[_R_W2:_R_W2 + 32, 0:64], a1T)
           + p_ref[_R_B2:_R_B2 + 32, 0:1])
    m2 = jnp.sum(h2T, axis=1, keepdims=True) * (1.0 / _B)
    v2 = jnp.sum(h2T * h2T, axis=1, keepdims=True) * (1.0 / _B) - m2 * m2
    sc2 = p_ref[_R_G2:_R_G2 + 32, 0:1] * jax.lax.rsqrt(v2 + _EPS)
    sh2 = p_ref[_R_BE2:_R_BE2 + 32, 0:1] - m2 * sc2
    a2T = jnp.maximum(h2T * sc2 + sh2, 0.0)

    dnnT = (jnp.dot(p_ref[_R_W3:_R_W3 + 64, 0:32], a2T)
            + p_ref[_R_B3:_R_B3 + 64, 0:1])

    wideT = jnp.dot(wwfT, xT) + p_ref[_R_BW:_R_BW + 64, 0:1]
    gluT = dnnT * jax.nn.sigmoid(wideT)
    logitT = (jnp.dot(p_ref[_R_WC:_R_WC + 1, 0:64], gluT)
              + p_ref[_R_BC:_R_BC + 1, 0:1])
    out_ref[...] = jax.nn.sigmoid(logitT)


@functools.partial(jax.jit)
def kernel(x, W_emb, b_emb, W1, b1, g1, be1, W2, b2, g2, be2, W3, b3,
           Ww, bw, Wc, bc):
    f32 = jnp.float32

    def _blk(a, rows):
        a = a.astype(f32)
        return jnp.pad(a, ((0, rows - a.shape[0]), (0, 128 - a.shape[1])))

    col = lambda v: v.reshape(-1, 1)
    p = jnp.concatenate([
        _blk(W_emb, 16), _blk(W1, 64), _blk(W2, 32), _blk(W3, 64),
        _blk(Ww, 64), _blk(Wc, 8),
        _blk(col(b_emb), 16), _blk(col(b1), 64), _blk(col(g1), 64),
        _blk(col(be1), 64), _blk(col(b2), 32), _blk(col(g2), 32),
        _blk(col(be2), 32), _blk(col(b3), 64), _blk(col(bw), 64),
        _blk(bc.reshape(1, 1), 8),
    ], axis=0)
    return pl.pallas_call(
        _fused_kernel,
        in_specs=[
            pl.BlockSpec((_B, _F), lambda: (0, 0)),
            pl.BlockSpec((_ROWS, 128), lambda: (0, 0)),
        ],
        out_specs=pl.BlockSpec((1, _B), lambda: (0, 0)),
        out_shape=jax.ShapeDtypeStruct((1, _B), f32),
    )(x.astype(f32), p).reshape(_B, 1)


# speedup vs baseline: 1.1766x; 1.0132x over previous
import functools

import jax
import jax.numpy as jnp
from jax.experimental import pallas as pl
from jax.experimental.pallas import tpu as pltpu

_B = 16384
_F = 13
_EPS = 1e-5

_NT = (((1,), (1,)), ((), ()))

_R_WEMB = 0
_R_W1 = 16
_R_W2 = 80
_R_W3 = 112
_R_WW = 176
_WROWS = 240

_L_BEMB = 0
_L_B1 = 128
_L_G1 = 256
_L_BE1 = 384
_L_B2 = 512
_L_G2 = 640
_L_BE2 = 768
_L_B3 = 896
_L_BW = 1024
_L_WC = 1152
_L_BC = 1280
_LANES = 1408


def _fused_kernel(x_ref, w_ref, bv_ref, out_ref):
    col = lambda off, n: jnp.transpose(bv_ref[0:1, off:off + n])
    xT = jnp.transpose(x_ref[...])

    embT = (jnp.dot(w_ref[_R_WEMB:_R_WEMB + _F, 0:_F], xT)
            + col(_L_BEMB, _F))
    dots = jnp.sum(embT * embT[0:1, :], axis=1, keepdims=True)
    nrm2 = jnp.sum(embT * embT, axis=1, keepdims=True)
    nrm = jnp.sqrt(nrm2)
    q = nrm[0:1, 0:1]
    dist = 1.0 - dots / (nrm * q + 1e-12)
    drow = jnp.transpose(dist)
    dself = jnp.broadcast_to(dist, (_F, _F))
    dk = jnp.broadcast_to(drow, (_F, _F))
    kk = jax.lax.broadcasted_iota(jnp.int32, (_F, _F), 1)
    jj = jax.lax.broadcasted_iota(jnp.int32, (_F, _F), 0)
    before = (dk < dself) | ((dk == dself) & (kk < jj))
    rank = jnp.sum(before.astype(jnp.float32), axis=1, keepdims=True)
    j0 = jax.lax.broadcasted_iota(jnp.int32, (_F, 1), 0)
    wide_m = ((rank >= float(_F - 6)) | (j0 == 0)).astype(jnp.float32)
    deep_m = 1.0 - wide_m
    lt = (kk < jj).astype(jnp.float32)
    pos_w = jnp.dot(lt, wide_m)
    pos_d = jnp.dot(lt, deep_m)
    cols7 = jax.lax.broadcasted_iota(jnp.int32, (_F, 7), 1).astype(
        jnp.float32)
    cols6 = jax.lax.broadcasted_iota(jnp.int32, (_F, 6), 1).astype(
        jnp.float32)
    s_wide = jnp.where(pos_w == cols7, wide_m, 0.0)
    s_deep = jnp.where(pos_d == cols6, deep_m, 0.0)
    w1dT = jax.lax.dot_general(
        w_ref[_R_W1:_R_W1 + 64, 0:6], s_deep, _NT)
    wwfT = jax.lax.dot_general(
        w_ref[_R_WW:_R_WW + 64, 0:7], s_wide, _NT)

    h1T = jnp.dot(w1dT, xT) + col(_L_B1, 64)
    m1 = jnp.sum(h1T, axis=1, keepdims=True) * (1.0 / _B)
    v1 = jnp.sum(h1T * h1T, axis=1, keepdims=True) * (1.0 / _B) - m1 * m1
    sc1 = col(_L_G1, 64) * jax.lax.rsqrt(v1 + _EPS)
    sh1 = col(_L_BE1, 64) - m1 * sc1
    a1T = jnp.maximum(h1T * sc1 + sh1, 0.0)

    h2T = jnp.dot(w_ref[_R_W2:_R_W2 + 32, 0:64], a1T) + col(_L_B2, 32)
    m2 = jnp.sum(h2T, axis=1, keepdims=True) * (1.0 / _B)
    v2 = jnp.sum(h2T * h2T, axis=1, keepdims=True) * (1.0 / _B) - m2 * m2
    sc2 = col(_L_G2, 32) * jax.lax.rsqrt(v2 + _EPS)
    sh2 = col(_L_BE2, 32) - m2 * sc2
    a2T = jnp.maximum(h2T * sc2 + sh2, 0.0)

    dnnT = jnp.dot(w_ref[_R_W3:_R_W3 + 64, 0:32], a2T) + col(_L_B3, 64)

    wideT = jnp.dot(wwfT, xT) + col(_L_BW, 64)
    gluT = dnnT * jax.nn.sigmoid(wideT)
    logitT = (jnp.dot(bv_ref[0:1, _L_WC:_L_WC + 64], gluT)
              + bv_ref[0:1, _L_BC:_L_BC + 1])
    out_ref[...] = jax.nn.sigmoid(logitT)


@functools.partial(jax.jit)
def kernel(x, W_emb, b_emb, W1, b1, g1, be1, W2, b2, g2, be2, W3, b3,
           Ww, bw, Wc, bc):
    f32 = jnp.float32

    def _blk(a, rows):
        a = a.astype(f32)
        return jnp.pad(a, ((0, rows - a.shape[0]), (0, 128 - a.shape[1])))

    w = jnp.concatenate([
        _blk(W_emb, 16), _blk(W1, 64), _blk(W2, 32), _blk(W3, 64),
        _blk(Ww, 64),
    ], axis=0)

    def _lane(v):
        v = v.reshape(-1).astype(f32)
        return jnp.pad(v, (0, 128 - v.shape[0]))

    bv = jnp.concatenate([
        _lane(b_emb), _lane(b1), _lane(g1), _lane(be1), _lane(b2),
        _lane(g2), _lane(be2), _lane(b3), _lane(bw), _lane(Wc),
        _lane(bc),
    ]).reshape(1, _LANES)

    return pl.pallas_call(
        _fused_kernel,
        in_specs=[
            pl.BlockSpec((_B, _F), lambda: (0, 0)),
            pl.BlockSpec((_WROWS, 128), lambda: (0, 0)),
            pl.BlockSpec((1, _LANES), lambda: (0, 0)),
        ],
        out_specs=pl.BlockSpec((1, _B), lambda: (0, 0)),
        out_shape=jax.ShapeDtypeStruct((1, _B), f32),
    )(x.astype(f32), w, bv).reshape(_B, 1)


# speedup vs baseline: 1.4431x; 1.2265x over previous
import functools

import jax
import jax.numpy as jnp
from jax.experimental import pallas as pl
from jax.experimental.pallas import tpu as pltpu

_B = 16384
_F = 13
_EPS = 1e-5

_NT = (((1,), (1,)), ((), ()))

_L_BEMB = 0
_L_B1 = 128
_L_G1 = 256
_L_BE1 = 384
_L_B2 = 512
_L_G2 = 640
_L_BE2 = 768
_L_B3 = 896
_L_BW = 1024
_L_WC = 1152
_L_BC = 1280
_LANES = 1408


def _fused_kernel(x_ref, wemb_ref, w1_ref, w2_ref, w3_ref, ww_ref, bv_ref,
                  out_ref):
    col = lambda off, n: jnp.transpose(bv_ref[0:1, off:off + n])
    xT = jnp.transpose(x_ref[...])

    embT = jnp.dot(wemb_ref[...], xT) + col(_L_BEMB, _F)
    dots = jnp.sum(embT * embT[0:1, :], axis=1, keepdims=True)
    nrm2 = jnp.sum(embT * embT, axis=1, keepdims=True)
    nrm = jnp.sqrt(nrm2)
    q = nrm[0:1, 0:1]
    dist = 1.0 - dots / (nrm * q + 1e-12)
    drow = jnp.transpose(dist)
    dself = jnp.broadcast_to(dist, (_F, _F))
    dk = jnp.broadcast_to(drow, (_F, _F))
    kk = jax.lax.broadcasted_iota(jnp.int32, (_F, _F), 1)
    jj = jax.lax.broadcasted_iota(jnp.int32, (_F, _F), 0)
    before = (dk < dself) | ((dk == dself) & (kk < jj))
    rank = jnp.sum(before.astype(jnp.float32), axis=1, keepdims=True)
    j0 = jax.lax.broadcasted_iota(jnp.int32, (_F, 1), 0)
    wide_m = ((rank >= float(_F - 6)) | (j0 == 0)).astype(jnp.float32)
    deep_m = 1.0 - wide_m
    lt = (kk < jj).astype(jnp.float32)
    pos_w = jnp.dot(lt, wide_m)
    pos_d = jnp.dot(lt, deep_m)
    cols7 = jax.lax.broadcasted_iota(jnp.int32, (_F, 7), 1).astype(
        jnp.float32)
    cols6 = jax.lax.broadcasted_iota(jnp.int32, (_F, 6), 1).astype(
        jnp.float32)
    s_wide = jnp.where(pos_w == cols7, wide_m, 0.0)
    s_deep = jnp.where(pos_d == cols6, deep_m, 0.0)
    w1dT = jax.lax.dot_general(w1_ref[...], s_deep, _NT)
    wwfT = jax.lax.dot_general(ww_ref[...], s_wide, _NT)

    h1T = jnp.dot(w1dT, xT) + col(_L_B1, 64)
    m1 = jnp.sum(h1T, axis=1, keepdims=True) * (1.0 / _B)
    v1 = jnp.sum(h1T * h1T, axis=1, keepdims=True) * (1.0 / _B) - m1 * m1
    sc1 = col(_L_G1, 64) * jax.lax.rsqrt(v1 + _EPS)
    sh1 = col(_L_BE1, 64) - m1 * sc1
    a1T = jnp.maximum(h1T * sc1 + sh1, 0.0)

    h2T = jnp.dot(w2_ref[...], a1T) + col(_L_B2, 32)
    m2 = jnp.sum(h2T, axis=1, keepdims=True) * (1.0 / _B)
    v2 = jnp.sum(h2T * h2T, axis=1, keepdims=True) * (1.0 / _B) - m2 * m2
    sc2 = col(_L_G2, 32) * jax.lax.rsqrt(v2 + _EPS)
    sh2 = col(_L_BE2, 32) - m2 * sc2
    a2T = jnp.maximum(h2T * sc2 + sh2, 0.0)

    dnnT = jnp.dot(w3_ref[...], a2T) + col(_L_B3, 64)

    wideT = jnp.dot(wwfT, xT) + col(_L_BW, 64)
    gluT = dnnT * jax.nn.sigmoid(wideT)
    logitT = (jnp.dot(bv_ref[0:1, _L_WC:_L_WC + 64], gluT)
              + bv_ref[0:1, _L_BC:_L_BC + 1])
    sig = jax.nn.sigmoid(logitT)
    out_ref[...] = jnp.reshape(sig, (128, 128))


@functools.partial(jax.jit)
def kernel(x, W_emb, b_emb, W1, b1, g1, be1, W2, b2, g2, be2, W3, b3,
           Ww, bw, Wc, bc):
    f32 = jnp.float32
    z115 = jnp.zeros((115,), f32)
    z64 = jnp.zeros((64,), f32)
    z96 = jnp.zeros((96,), f32)
    z127 = jnp.zeros((127,), f32)
    bv = jnp.concatenate([
        b_emb, z115, b1, z64, g1, z64, be1, z64, b2, z96, g2, z96,
        be2, z96, b3, z64, bw, z64, Wc.reshape(-1), z64, bc, z127,
    ]).reshape(1, _LANES)
    return pl.pallas_call(
        _fused_kernel,
        in_specs=[
            pl.BlockSpec((_B, _F), lambda: (0, 0)),
            pl.BlockSpec((_F, _F), lambda: (0, 0)),
            pl.BlockSpec((64, 6), lambda: (0, 0)),
            pl.BlockSpec((32, 64), lambda: (0, 0)),
            pl.BlockSpec((64, 32), lambda: (0, 0)),
            pl.BlockSpec((64, 7), lambda: (0, 0)),
            pl.BlockSpec((1, _LANES), lambda: (0, 0)),
        ],
        out_specs=pl.BlockSpec((128, 128), lambda: (0, 0)),
        out_shape=jax.ShapeDtypeStruct((128, 128), f32),
    )(x, W_emb, W1, W2, W3, Ww, bv).reshape(_B, 1)


# speedup vs baseline: 3.2230x; 2.2334x over previous
import functools

import jax
import jax.numpy as jnp
from jax.experimental import pallas as pl
from jax.experimental.pallas import tpu as pltpu

_B = 16384
_F = 13
_EPS = 1e-5

_TN = (((0,), (0,)), ((), ()))
_T1 = (((0,), (1,)), ((), ()))

_L_BEMB = 0
_L_B1 = 128
_L_G1 = 256
_L_BE1 = 384
_L_B2 = 512
_L_G2 = 640
_L_BE2 = 768
_L_B3 = 896
_L_BW = 1024
_L_WC = 1152
_L_BC = 1280
_LANES = 1408


def _fused_kernel(xT_ref, wembT_ref, w1T_ref, w2T_ref, w3T_ref, wwT_ref,
                  bv_ref, out_ref):
    col = lambda off, n: jnp.transpose(bv_ref[0:1, off:off + n])
    xT = xT_ref[...]

    embT = (jax.lax.dot_general(wembT_ref[...], xT, _TN)
            + col(_L_BEMB, _F))
    dots = jnp.sum(embT * embT[0:1, :], axis=1, keepdims=True)
    nrm2 = jnp.sum(embT * embT, axis=1, keepdims=True)
    nrm = jnp.sqrt(nrm2)
    q = nrm[0:1, 0:1]
    dist = 1.0 - dots / (nrm * q + 1e-12)
    drow = jnp.transpose(dist)
    dself = jnp.broadcast_to(dist, (_F, _F))
    dk = jnp.broadcast_to(drow, (_F, _F))
    kk = jax.lax.broadcasted_iota(jnp.int32, (_F, _F), 1)
    jj = jax.lax.broadcasted_iota(jnp.int32, (_F, _F), 0)
    before = (dk < dself) | ((dk == dself) & (kk < jj))
    rank = jnp.sum(before.astype(jnp.float32), axis=1, keepdims=True)
    j0 = jax.lax.broadcasted_iota(jnp.int32, (_F, 1), 0)
    wide_m = ((rank >= float(_F - 6)) | (j0 == 0)).astype(jnp.float32)
    deep_m = 1.0 - wide_m
    lt = (kk < jj).astype(jnp.float32)
    pos_w = jnp.dot(lt, wide_m)
    pos_d = jnp.dot(lt, deep_m)
    cols7 = jax.lax.broadcasted_iota(jnp.int32, (_F, 7), 1).astype(
        jnp.float32)
    cols6 = jax.lax.broadcasted_iota(jnp.int32, (_F, 6), 1).astype(
        jnp.float32)
    s_wide = jnp.where(pos_w == cols7, wide_m, 0.0)
    s_deep = jnp.where(pos_d == cols6, deep_m, 0.0)
    w1dT = jax.lax.dot_general(w1T_ref[...], s_deep, _T1)
    wwfT = jax.lax.dot_general(wwT_ref[...], s_wide, _T1)

    h1T = jnp.dot(w1dT, xT) + col(_L_B1, 64)
    m1 = jnp.sum(h1T, axis=1, keepdims=True) * (1.0 / _B)
    v1 = jnp.sum(h1T * h1T, axis=1, keepdims=True) * (1.0 / _B) - m1 * m1
    sc1 = col(_L_G1, 64) * jax.lax.rsqrt(v1 + _EPS)
    sh1 = col(_L_BE1, 64) - m1 * sc1
    a1T = jnp.maximum(h1T * sc1 + sh1, 0.0)

    h2T = (jax.lax.dot_general(w2T_ref[...], a1T, _TN)
           + col(_L_B2, 32))
    m2 = jnp.sum(h2T, axis=1, keepdims=True) * (1.0 / _B)
    v2 = jnp.sum(h2T * h2T, axis=1, keepdims=True) * (1.0 / _B) - m2 * m2
    sc2 = col(_L_G2, 32) * jax.lax.rsqrt(v2 + _EPS)
    sh2 = col(_L_BE2, 32) - m2 * sc2
    a2T = jnp.maximum(h2T * sc2 + sh2, 0.0)

    dnnT = (jax.lax.dot_general(w3T_ref[...], a2T, _TN)
            + col(_L_B3, 64))

    wideT = jnp.dot(wwfT, xT) + col(_L_BW, 64)
    gluT = dnnT * jax.nn.sigmoid(wideT)
    logitT = (jnp.dot(bv_ref[0:1, _L_WC:_L_WC + 64], gluT)
              + bv_ref[0:1, _L_BC:_L_BC + 1])
    sig = jax.nn.sigmoid(logitT)
    out_ref[...] = jnp.reshape(sig, (128, 128))


@functools.partial(jax.jit)
def kernel(x, W_emb, b_emb, W1, b1, g1, be1, W2, b2, g2, be2, W3, b3,
           Ww, bw, Wc, bc):
    f32 = jnp.float32
    z115 = jnp.zeros((115,), f32)
    z64 = jnp.zeros((64,), f32)
    z96 = jnp.zeros((96,), f32)
    z127 = jnp.zeros((127,), f32)
    bv = jnp.concatenate([
        b_emb, z115, b1, z64, g1, z64, be1, z64, b2, z96, g2, z96,
        be2, z96, b3, z64, bw, z64, Wc.reshape(-1), z64, bc, z127,
    ]).reshape(1, _LANES)
    return pl.pallas_call(
        _fused_kernel,
        in_specs=[
            pl.BlockSpec((_F, _B), lambda: (0, 0)),
            pl.BlockSpec((_F, _F), lambda: (0, 0)),
            pl.BlockSpec((6, 64), lambda: (0, 0)),
            pl.BlockSpec((64, 32), lambda: (0, 0)),
            pl.BlockSpec((32, 64), lambda: (0, 0)),
            pl.BlockSpec((7, 64), lambda: (0, 0)),
            pl.BlockSpec((1, _LANES), lambda: (0, 0)),
        ],
        out_specs=pl.BlockSpec((128, 128), lambda: (0, 0)),
        out_shape=jax.ShapeDtypeStruct((128, 128), f32),
    )(x.T, W_emb.T, W1.T, W2.T, W3.T, Ww.T, bv).reshape(_B, 1)
